# Initial kernel scaffold; baseline (speedup 1.0000x reference)
import jax
import jax.numpy as jnp
from jax.experimental import pallas as pl


def kernel(x, edges, W, b):
    raise NotImplementedError("write your pallas kernel here")


# speedup vs baseline: 33.2422x; 33.2422x over previous
import functools

import jax
import jax.numpy as jnp
from jax import lax
from jax.experimental import pallas as pl
from jax.experimental.pallas import tpu as pltpu
from jax.experimental.pallas import tpu_sc as plsc

N_NODES = 10000
N_EDGES = 320000
D_IN = 128
D_OUT = 64
HALF = D_OUT // 2
NSC = 2
NSUB = 16
SLICE = 640
N_PAD = NSUB * SLICE
WIN = 1024
WROWS = WIN // 128
EDGES_PER_TILE = 20480
E_PAD = NSUB * EDGES_PER_TILE
NWIN = EDGES_PER_TILE // WIN
NB = 2560


def _mm_body(x_ref, w_ref, b_ref, o_ref):
    o_ref[0] = (
        lax.dot_general(x_ref[...], w_ref[0], (((1,), (1,)), ((), ())),
                        preferred_element_type=jnp.float32)
        + b_ref[0]
    )


def _linear_tc(x_pad, W, b):
    return pl.pallas_call(
        _mm_body,
        grid=(NSC, N_PAD // NB),
        in_specs=[
            pl.BlockSpec((NB, D_IN), lambda c, n: (n, 0)),
            pl.BlockSpec((1, HALF, D_IN), lambda c, n: (c, 0, 0)),
            pl.BlockSpec((1, 1, HALF), lambda c, n: (c, 0, 0)),
        ],
        out_specs=pl.BlockSpec((1, NB, HALF), lambda c, n: (c, n, 0)),
        out_shape=jax.ShapeDtypeStruct((NSC, N_PAD, HALF), jnp.float32),
    )(x_pad, W.reshape(NSC, HALF, D_IN), b.reshape(NSC, 1, HALF))


_MESH = plsc.VectorSubcoreMesh(
    core_axis_name="c", subcore_axis_name="s", num_cores=NSC, num_subcores=NSUB
)


@functools.partial(
    pl.kernel,
    out_type=jax.ShapeDtypeStruct((NSC, N_PAD, HALF), jnp.float32),
    mesh=_MESH,
    compiler_params=pltpu.CompilerParams(use_tc_tiling_on_sc=False),
    scratch_types=[
        pltpu.VMEM_SHARED((N_PAD, HALF), jnp.float32),
        pltpu.VMEM_SHARED((N_PAD, HALF), jnp.float32),
        pltpu.VMEM_SHARED((N_PAD,), jnp.float32),
        pltpu.VMEM((SLICE, HALF), jnp.float32),
        pltpu.VMEM((SLICE,), jnp.float32),
        pltpu.VMEM((WIN,), jnp.float32),
        pltpu.VMEM((WROWS, 128), jnp.int32),
        pltpu.VMEM((WROWS, 128), jnp.int32),
        pltpu.VMEM((WIN, HALF), jnp.float32),
        pltpu.SemaphoreType.DMA,
    ],
)
def _sgc_sc(row_hbm, col_hbm, h0_hbm, out_hbm,
            g_sh, acc_sh, deg_sh, wbuf, dbuf, ones, ridx, cidx, gbuf, sem):
    s = lax.axis_index("s")
    c = lax.axis_index("c")
    r0 = SLICE * s
    ebase = s * (EDGES_PER_TILE // 128)

    def _za(i, carry):
        dbuf[pl.ds(i * 16, 16)] = jnp.zeros((16,), jnp.float32)
        return carry
    lax.fori_loop(0, SLICE // 16, _za, 0)

    def _zb(i, carry):
        ones[pl.ds(i * 16, 16)] = jnp.ones((16,), jnp.float32)
        return carry
    lax.fori_loop(0, WIN // 16, _zb, 0)

    pltpu.sync_copy(dbuf, deg_sh.at[pl.ds(r0, SLICE)])
    plsc.subcore_barrier()

    def _deg_win(w, carry):
        rb = ebase + w * WROWS
        pltpu.sync_copy(col_hbm.at[pl.ds(rb, WROWS)], cidx)
        for j in range(WROWS):
            pltpu.sync_copy(ones.at[pl.ds(j * 128, 128)],
                            deg_sh.at[cidx.at[j]], add=True)
        return carry
    lax.fori_loop(0, NWIN, _deg_win, 0)
    plsc.subcore_barrier()

    pltpu.sync_copy(deg_sh.at[pl.ds(r0, SLICE)], dbuf)

    def _rsqrt(i, carry):
        d = dbuf[pl.ds(i * 16, 16)] + 1.0
        st = d
        for _ in range(24):
            st = 0.5 * (st + d / st)
        dbuf[pl.ds(i * 16, 16)] = 1.0 / st
        return carry
    lax.fori_loop(0, SLICE // 16, _rsqrt, 0)

    pltpu.sync_copy(h0_hbm.at[c, pl.ds(r0, SLICE), :], wbuf)

    def _make_scale(square):
        def _scale(i, carry):
            dvec = dbuf[pl.ds(i * 16, 16)]
            if square:
                dvec = dvec * dvec
            for r in range(16):
                dv = jnp.full((16,), dvec[r])
                base = i * 16 + r
                wbuf[base, pl.ds(0, 16)] = wbuf[base, pl.ds(0, 16)] * dv
                wbuf[base, pl.ds(16, 16)] = wbuf[base, pl.ds(16, 16)] * dv
            return carry
        return _scale

    lax.fori_loop(0, SLICE // 16, _make_scale(False), 0)

    pltpu.sync_copy(wbuf, g_sh.at[pl.ds(r0, SLICE)])
    pltpu.sync_copy(wbuf, acc_sh.at[pl.ds(r0, SLICE)])
    plsc.subcore_barrier()

    for rnd in range(2):
        def _win(w, carry):
            rb = ebase + w * WROWS
            pltpu.sync_copy(row_hbm.at[pl.ds(rb, WROWS)], ridx)
            pltpu.sync_copy(col_hbm.at[pl.ds(rb, WROWS)], cidx)
            cps = [
                pltpu.async_copy(g_sh.at[ridx.at[j]],
                                 gbuf.at[pl.ds(j * 128, 128)], sem)
                for j in range(WROWS)
            ]
            for cp in cps:
                cp.wait()
            for j in range(WROWS):
                pltpu.sync_copy(gbuf.at[pl.ds(j * 128, 128)],
                                acc_sh.at[cidx.at[j]], add=True)
            return carry
        lax.fori_loop(0, NWIN, _win, 0)
        plsc.subcore_barrier()

        pltpu.sync_copy(acc_sh.at[pl.ds(r0, SLICE)], wbuf)
        if rnd == 0:
            lax.fori_loop(0, SLICE // 16, _make_scale(True), 0)
            pltpu.sync_copy(wbuf, g_sh.at[pl.ds(r0, SLICE)])
            pltpu.sync_copy(wbuf, acc_sh.at[pl.ds(r0, SLICE)])
            plsc.subcore_barrier()
        else:
            lax.fori_loop(0, SLICE // 16, _make_scale(False), 0)
            pltpu.sync_copy(wbuf, out_hbm.at[c, pl.ds(r0, SLICE), :])


def kernel(x, edges, W, b):
    row = edges[0].astype(jnp.int32)
    col = edges[1].astype(jnp.int32)
    pad_n = E_PAD - N_EDGES
    dump = N_NODES + (jnp.arange(pad_n, dtype=jnp.int32) % (N_PAD - N_NODES))
    row_p = jnp.concatenate([row, dump]).reshape(E_PAD // 128, 128)
    col_p = jnp.concatenate([col, dump]).reshape(E_PAD // 128, 128)

    x_pad = jnp.pad(x, ((0, N_PAD - N_NODES), (0, 0)))
    h0 = _linear_tc(x_pad, W, b)

    out = _sgc_sc(row_p, col_p, h0)
    return jnp.concatenate([out[0, :N_NODES], out[1, :N_NODES]], axis=1)


# speedup vs baseline: 36.9794x; 1.1124x over previous
import functools

import jax
import jax.numpy as jnp
from jax import lax
from jax.experimental import pallas as pl
from jax.experimental.pallas import tpu as pltpu
from jax.experimental.pallas import tpu_sc as plsc

N_NODES = 10000
N_EDGES = 320000
D_IN = 128
D_OUT = 64
HALF = D_OUT // 2
NSC = 2
NSUB = 16
SLICE = 640
N_PAD = NSUB * SLICE
WIN = 1024
WROWS = WIN // 128
EDGES_PER_TILE = 20480
E_PAD = NSUB * EDGES_PER_TILE
NWIN = EDGES_PER_TILE // WIN
NPAIR = NWIN // 2
NB = 2560


def _mm_body(x_ref, w_ref, b_ref, o_ref):
    o_ref[0] = (
        lax.dot_general(x_ref[...], w_ref[0], (((1,), (1,)), ((), ())),
                        preferred_element_type=jnp.float32)
        + b_ref[0]
    )


def _linear_tc(x_pad, W, b):
    return pl.pallas_call(
        _mm_body,
        grid=(NSC, N_PAD // NB),
        in_specs=[
            pl.BlockSpec((NB, D_IN), lambda c, n: (n, 0)),
            pl.BlockSpec((1, HALF, D_IN), lambda c, n: (c, 0, 0)),
            pl.BlockSpec((1, 1, HALF), lambda c, n: (c, 0, 0)),
        ],
        out_specs=pl.BlockSpec((1, NB, HALF), lambda c, n: (c, n, 0)),
        out_shape=jax.ShapeDtypeStruct((NSC, N_PAD, HALF), jnp.float32),
    )(x_pad, W.reshape(NSC, HALF, D_IN), b.reshape(NSC, 1, HALF))


_MESH = plsc.VectorSubcoreMesh(
    core_axis_name="c", subcore_axis_name="s", num_cores=NSC, num_subcores=NSUB
)


@functools.partial(
    pl.kernel,
    out_type=jax.ShapeDtypeStruct((NSC, N_PAD, HALF), jnp.float32),
    mesh=_MESH,
    compiler_params=pltpu.CompilerParams(use_tc_tiling_on_sc=False),
    scratch_types=[
        pltpu.VMEM_SHARED((N_PAD, HALF), jnp.float32),
        pltpu.VMEM_SHARED((N_PAD, HALF), jnp.float32),
        pltpu.VMEM_SHARED((N_PAD,), jnp.float32),
        pltpu.VMEM((SLICE,), jnp.float32),
        pltpu.VMEM((128,), jnp.float32),
        pltpu.VMEM((WROWS, 128), jnp.int32),
        pltpu.VMEM((WROWS, 128), jnp.int32),
        pltpu.VMEM((WROWS, 128), jnp.int32),
        pltpu.VMEM((WROWS, 128), jnp.int32),
        pltpu.VMEM((WIN, HALF), jnp.float32),
        pltpu.VMEM((WIN, HALF), jnp.float32),
        pltpu.SemaphoreType.DMA,
        pltpu.SemaphoreType.DMA,
    ],
)
def _sgc_sc(row_hbm, col_hbm, h0_hbm, out_hbm,
            g_sh, acc_sh, deg_sh, dbuf, ones,
            ridxA, cidxA, ridxB, cidxB, gbufA, gbufB, gsem, ssem):
    s = lax.axis_index("s")
    c = lax.axis_index("c")
    r0 = SLICE * s
    ebase = s * (EDGES_PER_TILE // 128)
    wslc = gbufA.at[pl.ds(0, SLICE), :]

    def _widx(w):
        return ebase + w * WROWS

    def _za(i, carry):
        dbuf[pl.ds(i * 16, 16)] = jnp.zeros((16,), jnp.float32)
        return carry
    lax.fori_loop(0, SLICE // 16, _za, 0)
    for i in range(128 // 16):
        ones[pl.ds(i * 16, 16)] = jnp.ones((16,), jnp.float32)

    pltpu.sync_copy(dbuf, deg_sh.at[pl.ds(r0, SLICE)])
    plsc.subcore_barrier()

    pltpu.sync_copy(col_hbm.at[pl.ds(_widx(0), WROWS)], cidxA)

    def _deg_pair(k, carry):
        w0 = 2 * k
        sc0 = [pltpu.async_copy(ones, deg_sh.at[cidxA.at[j]], ssem, add=True)
               for j in range(WROWS)]
        pltpu.sync_copy(col_hbm.at[pl.ds(_widx(w0 + 1), WROWS)], cidxB)
        for cp in sc0:
            cp.wait()
        sc1 = [pltpu.async_copy(ones, deg_sh.at[cidxB.at[j]], ssem, add=True)
               for j in range(WROWS)]

        @pl.when(k < NPAIR - 1)
        def _():
            pltpu.sync_copy(col_hbm.at[pl.ds(_widx(w0 + 2), WROWS)], cidxA)
        for cp in sc1:
            cp.wait()
        return carry
    lax.fori_loop(0, NPAIR, _deg_pair, 0)
    plsc.subcore_barrier()

    pltpu.sync_copy(deg_sh.at[pl.ds(r0, SLICE)], dbuf)

    def _rsqrt(i, carry):
        d = dbuf[pl.ds(i * 16, 16)] + 1.0
        st = d
        for _ in range(24):
            st = 0.5 * (st + d / st)
        dbuf[pl.ds(i * 16, 16)] = 1.0 / st
        return carry
    lax.fori_loop(0, SLICE // 16, _rsqrt, 0)

    def _make_scale(square):
        def _scale(i, carry):
            dvec = dbuf[pl.ds(i * 16, 16)]
            if square:
                dvec = dvec * dvec
            for r in range(16):
                dv = jnp.full((16,), dvec[r])
                base = i * 16 + r
                wslc[base, pl.ds(0, 16)] = wslc[base, pl.ds(0, 16)] * dv
                wslc[base, pl.ds(16, 16)] = wslc[base, pl.ds(16, 16)] * dv
            return carry
        return _scale

    pltpu.sync_copy(h0_hbm.at[c, pl.ds(r0, SLICE), :], wslc)
    lax.fori_loop(0, SLICE // 16, _make_scale(False), 0)
    pltpu.sync_copy(wslc, g_sh.at[pl.ds(r0, SLICE)])
    pltpu.sync_copy(wslc, acc_sh.at[pl.ds(r0, SLICE)])
    plsc.subcore_barrier()

    def _fire_gathers(ridx, gbuf):
        for j in range(WROWS):
            pltpu.async_copy(g_sh.at[ridx.at[j]],
                             gbuf.at[pl.ds(j * 128, 128)], gsem)

    def _drain_gathers(ridx, gbuf):
        for j in range(WROWS):
            pltpu.make_async_copy(g_sh.at[ridx.at[j]],
                                  gbuf.at[pl.ds(j * 128, 128)], gsem).wait()

    def _load_idx(w, ridx, cidx):
        pltpu.sync_copy(row_hbm.at[pl.ds(_widx(w), WROWS)], ridx)
        pltpu.sync_copy(col_hbm.at[pl.ds(_widx(w), WROWS)], cidx)

    for rnd in range(2):
        _load_idx(0, ridxA, cidxA)
        _fire_gathers(ridxA, gbufA)

        def _pair(k, carry):
            w0 = 2 * k
            _load_idx(w0 + 1, ridxB, cidxB)
            _drain_gathers(ridxA, gbufA)
            sc0 = [pltpu.async_copy(gbufA.at[pl.ds(j * 128, 128)],
                                    acc_sh.at[cidxA.at[j]], ssem, add=True)
                   for j in range(WROWS)]
            _fire_gathers(ridxB, gbufB)
            for cp in sc0:
                cp.wait()

            @pl.when(k < NPAIR - 1)
            def _():
                _load_idx(w0 + 2, ridxA, cidxA)
            _drain_gathers(ridxB, gbufB)
            sc1 = [pltpu.async_copy(gbufB.at[pl.ds(j * 128, 128)],
                                    acc_sh.at[cidxB.at[j]], ssem, add=True)
                   for j in range(WROWS)]

            @pl.when(k < NPAIR - 1)
            def _():
                _fire_gathers(ridxA, gbufA)
            for cp in sc1:
                cp.wait()
            return carry
        lax.fori_loop(0, NPAIR, _pair, 0)
        plsc.subcore_barrier()

        pltpu.sync_copy(acc_sh.at[pl.ds(r0, SLICE)], wslc)
        if rnd == 0:
            lax.fori_loop(0, SLICE // 16, _make_scale(True), 0)
            pltpu.sync_copy(wslc, g_sh.at[pl.ds(r0, SLICE)])
            pltpu.sync_copy(wslc, acc_sh.at[pl.ds(r0, SLICE)])
            plsc.subcore_barrier()
        else:
            lax.fori_loop(0, SLICE // 16, _make_scale(False), 0)
            pltpu.sync_copy(wslc, out_hbm.at[c, pl.ds(r0, SLICE), :])


def kernel(x, edges, W, b):
    row = edges[0].astype(jnp.int32)
    col = edges[1].astype(jnp.int32)
    pad_n = E_PAD - N_EDGES
    dump = N_NODES + (jnp.arange(pad_n, dtype=jnp.int32) % (N_PAD - N_NODES))
    row_p = jnp.concatenate([row, dump]).reshape(E_PAD // 128, 128)
    col_p = jnp.concatenate([col, dump]).reshape(E_PAD // 128, 128)

    x_pad = jnp.pad(x, ((0, N_PAD - N_NODES), (0, 0)))
    h0 = _linear_tc(x_pad, W, b)

    out = _sgc_sc(row_p, col_p, h0)
    return jnp.concatenate([out[0, :N_NODES], out[1, :N_NODES]], axis=1)


# speedup vs baseline: 37.3355x; 1.0096x over previous
import functools

import jax
import jax.numpy as jnp
from jax import lax
from jax.experimental import pallas as pl
from jax.experimental.pallas import tpu as pltpu
from jax.experimental.pallas import tpu_sc as plsc

N_NODES = 10000
N_EDGES = 320000
D_IN = 128
D_OUT = 64
HALF = D_OUT // 2
NSC = 2
NSUB = 16
SLICE = 640
N_PAD = NSUB * SLICE
WIN = 1024
WROWS = WIN // 128
EDGES_PER_TILE = 20480
E_PAD = NSUB * EDGES_PER_TILE
NWIN = EDGES_PER_TILE // WIN
NPAIR = NWIN // 2
NB = 2560


def _mm_body(x_ref, w_ref, b_ref, o_ref):
    o_ref[0] = (
        lax.dot_general(x_ref[...], w_ref[0], (((1,), (1,)), ((), ())),
                        preferred_element_type=jnp.float32)
        + b_ref[0]
    )


def _linear_tc(x_pad, W, b):
    return pl.pallas_call(
        _mm_body,
        grid=(NSC, N_PAD // NB),
        in_specs=[
            pl.BlockSpec((NB, D_IN), lambda c, n: (n, 0)),
            pl.BlockSpec((1, HALF, D_IN), lambda c, n: (c, 0, 0)),
            pl.BlockSpec((1, 1, HALF), lambda c, n: (c, 0, 0)),
        ],
        out_specs=pl.BlockSpec((1, NB, HALF), lambda c, n: (c, n, 0)),
        out_shape=jax.ShapeDtypeStruct((NSC, N_PAD, HALF), jnp.float32),
    )(x_pad, W.reshape(NSC, HALF, D_IN), b.reshape(NSC, 1, HALF))


_MESH = plsc.VectorSubcoreMesh(
    core_axis_name="c", subcore_axis_name="s", num_cores=NSC, num_subcores=NSUB
)


@functools.partial(
    pl.kernel,
    out_type=jax.ShapeDtypeStruct((NSC, N_PAD, HALF), jnp.float32),
    mesh=_MESH,
    compiler_params=pltpu.CompilerParams(use_tc_tiling_on_sc=False),
    scratch_types=[
        pltpu.VMEM_SHARED((N_PAD, HALF), jnp.float32),
        pltpu.VMEM_SHARED((N_PAD, HALF), jnp.float32),
        pltpu.VMEM_SHARED((N_PAD,), jnp.float32),
        pltpu.VMEM((SLICE,), jnp.float32),
        pltpu.VMEM((128,), jnp.float32),
        pltpu.VMEM((WROWS, 128), jnp.int32),
        pltpu.VMEM((WROWS, 128), jnp.int32),
        pltpu.VMEM((WROWS, 128), jnp.int32),
        pltpu.VMEM((WROWS, 128), jnp.int32),
        pltpu.VMEM((WIN, HALF), jnp.float32),
        pltpu.VMEM((WIN, HALF), jnp.float32),
        pltpu.SemaphoreType.DMA,
        pltpu.SemaphoreType.DMA,
    ],
)
def _sgc_sc(row_hbm, col_hbm, h0_hbm, out_hbm,
            g_sh, acc_sh, deg_sh, dbuf, ones,
            ridxA, cidxA, ridxB, cidxB, gbufA, gbufB, gsem, ssem):
    s = lax.axis_index("s")
    c = lax.axis_index("c")
    r0 = SLICE * s
    ebase = s * (EDGES_PER_TILE // 128)
    wslc = gbufA.at[pl.ds(0, SLICE), :]

    def _widx(w):
        return ebase + w * WROWS

    def _za(i, carry):
        dbuf[pl.ds(i * 16, 16)] = jnp.zeros((16,), jnp.float32)
        return carry
    lax.fori_loop(0, SLICE // 16, _za, 0)
    for i in range(128 // 16):
        ones[pl.ds(i * 16, 16)] = jnp.ones((16,), jnp.float32)

    pltpu.sync_copy(dbuf, deg_sh.at[pl.ds(r0, SLICE)])
    plsc.subcore_barrier()

    scope_deg = jax.named_scope("deg_phase")
    scope_deg.__enter__()
    pltpu.sync_copy(col_hbm.at[pl.ds(_widx(0), WROWS)], cidxA)

    def _deg_pair(k, carry):
        w0 = 2 * k
        sc0 = [pltpu.async_copy(ones, deg_sh.at[cidxA.at[j]], ssem, add=True)
               for j in range(WROWS)]
        pltpu.sync_copy(col_hbm.at[pl.ds(_widx(w0 + 1), WROWS)], cidxB)
        for cp in sc0:
            cp.wait()
        sc1 = [pltpu.async_copy(ones, deg_sh.at[cidxB.at[j]], ssem, add=True)
               for j in range(WROWS)]

        @pl.when(k < NPAIR - 1)
        def _():
            pltpu.sync_copy(col_hbm.at[pl.ds(_widx(w0 + 2), WROWS)], cidxA)
        for cp in sc1:
            cp.wait()
        return carry
    lax.fori_loop(0, NPAIR, _deg_pair, 0)
    plsc.subcore_barrier()
    scope_deg.__exit__(None, None, None)

    pltpu.sync_copy(deg_sh.at[pl.ds(r0, SLICE)], dbuf)

    def _rsqrt(i, carry):
        d = dbuf[pl.ds(i * 16, 16)] + 1.0
        st = d
        for _ in range(24):
            st = 0.5 * (st + d / st)
        dbuf[pl.ds(i * 16, 16)] = 1.0 / st
        return carry
    lax.fori_loop(0, SLICE // 16, _rsqrt, 0)

    def _make_scale(square):
        def _scale(i, carry):
            dvec = dbuf[pl.ds(i * 16, 16)]
            if square:
                dvec = dvec * dvec
            for r in range(16):
                dv = jnp.full((16,), dvec[r])
                base = i * 16 + r
                wslc[base, pl.ds(0, 16)] = wslc[base, pl.ds(0, 16)] * dv
                wslc[base, pl.ds(16, 16)] = wslc[base, pl.ds(16, 16)] * dv
            return carry
        return _scale

    pltpu.sync_copy(h0_hbm.at[c, pl.ds(r0, SLICE), :], wslc)
    lax.fori_loop(0, SLICE // 16, _make_scale(False), 0)
    pltpu.sync_copy(wslc, g_sh.at[pl.ds(r0, SLICE)])
    pltpu.sync_copy(wslc, acc_sh.at[pl.ds(r0, SLICE)])
    plsc.subcore_barrier()

    def _fire_gathers(ridx, gbuf):
        for j in range(WROWS):
            pltpu.async_copy(g_sh.at[ridx.at[j]],
                             gbuf.at[pl.ds(j * 128, 128)], gsem)

    def _drain_gathers(ridx, gbuf):
        for j in range(WROWS):
            pltpu.make_async_copy(g_sh.at[ridx.at[j]],
                                  gbuf.at[pl.ds(j * 128, 128)], gsem).wait()

    def _load_idx(w, ridx, cidx):
        pltpu.sync_copy(row_hbm.at[pl.ds(_widx(w), WROWS)], ridx)
        pltpu.sync_copy(col_hbm.at[pl.ds(_widx(w), WROWS)], cidx)

    for rnd in range(2):
        scope_rnd = jax.named_scope(f"round{rnd}")
        scope_rnd.__enter__()
        _load_idx(0, ridxA, cidxA)
        _fire_gathers(ridxA, gbufA)

        def _pair(k, carry):
            w0 = 2 * k
            _load_idx(w0 + 1, ridxB, cidxB)
            _drain_gathers(ridxA, gbufA)
            sc0 = [pltpu.async_copy(gbufA.at[pl.ds(j * 128, 128)],
                                    acc_sh.at[cidxA.at[j]], ssem, add=True)
                   for j in range(WROWS)]
            _fire_gathers(ridxB, gbufB)
            for cp in sc0:
                cp.wait()

            @pl.when(k < NPAIR - 1)
            def _():
                _load_idx(w0 + 2, ridxA, cidxA)
            _drain_gathers(ridxB, gbufB)
            sc1 = [pltpu.async_copy(gbufB.at[pl.ds(j * 128, 128)],
                                    acc_sh.at[cidxB.at[j]], ssem, add=True)
                   for j in range(WROWS)]

            @pl.when(k < NPAIR - 1)
            def _():
                _fire_gathers(ridxA, gbufA)
            for cp in sc1:
                cp.wait()
            return carry
        lax.fori_loop(0, NPAIR, _pair, 0)
        plsc.subcore_barrier()
        scope_rnd.__exit__(None, None, None)

        pltpu.sync_copy(acc_sh.at[pl.ds(r0, SLICE)], wslc)
        if rnd == 0:
            lax.fori_loop(0, SLICE // 16, _make_scale(True), 0)
            pltpu.sync_copy(wslc, g_sh.at[pl.ds(r0, SLICE)])
            pltpu.sync_copy(wslc, acc_sh.at[pl.ds(r0, SLICE)])
            plsc.subcore_barrier()
        else:
            lax.fori_loop(0, SLICE // 16, _make_scale(False), 0)
            pltpu.sync_copy(wslc, out_hbm.at[c, pl.ds(r0, SLICE), :])


def kernel(x, edges, W, b):
    row = edges[0].astype(jnp.int32)
    col = edges[1].astype(jnp.int32)
    pad_n = E_PAD - N_EDGES
    dump = N_NODES + (jnp.arange(pad_n, dtype=jnp.int32) % (N_PAD - N_NODES))
    row_p = jnp.concatenate([row, dump]).reshape(E_PAD // 128, 128)
    col_p = jnp.concatenate([col, dump]).reshape(E_PAD // 128, 128)

    x_pad = jnp.pad(x, ((0, N_PAD - N_NODES), (0, 0)))
    h0 = _linear_tc(x_pad, W, b)

    out = _sgc_sc(row_p, col_p, h0)
    return jnp.concatenate([out[0, :N_NODES], out[1, :N_NODES]], axis=1)


# speedup vs baseline: 41.7623x; 1.1186x over previous
import functools

import jax
import jax.numpy as jnp
from jax import lax
from jax.experimental import pallas as pl
from jax.experimental.pallas import tpu as pltpu
from jax.experimental.pallas import tpu_sc as plsc

N_NODES = 10000
N_EDGES = 320000
D_IN = 128
D_OUT = 64
HALF = D_OUT // 2
NSC = 2
NSUB = 16
SLICE = 640
N_PAD = NSUB * SLICE
TAIL = N_NODES - (NSUB - 1) * SLICE
WIN = 1024
WROWS = WIN // 128
EDGES_PER_TILE = 20480
E_PAD = NSUB * EDGES_PER_TILE
NWIN = EDGES_PER_TILE // WIN
NPAIR = NWIN // 2
NB = 2000


def _mm_body(x_ref, w_ref, b_ref, o_ref):
    o_ref[0] = (
        lax.dot_general(x_ref[...], w_ref[0], (((1,), (1,)), ((), ())),
                        preferred_element_type=jnp.float32)
        + b_ref[0]
    )


def _linear_tc(x, W, b):
    return pl.pallas_call(
        _mm_body,
        grid=(NSC, N_NODES // NB),
        in_specs=[
            pl.BlockSpec((NB, D_IN), lambda c, n: (n, 0)),
            pl.BlockSpec((1, HALF, D_IN), lambda c, n: (c, 0, 0)),
            pl.BlockSpec((1, 1, HALF), lambda c, n: (c, 0, 0)),
        ],
        out_specs=pl.BlockSpec((1, NB, HALF), lambda c, n: (c, n, 0)),
        out_shape=jax.ShapeDtypeStruct((NSC, N_NODES, HALF), jnp.float32),
    )(x, W.reshape(NSC, HALF, D_IN), b.reshape(NSC, 1, HALF))


_MESH = plsc.VectorSubcoreMesh(
    core_axis_name="c", subcore_axis_name="s", num_cores=NSC, num_subcores=NSUB
)


@functools.partial(
    pl.kernel,
    out_type=jax.ShapeDtypeStruct((N_PAD,), jnp.float32),
    mesh=_MESH,
    compiler_params=pltpu.CompilerParams(use_tc_tiling_on_sc=False),
    scratch_types=[
        pltpu.VMEM_SHARED((N_PAD,), jnp.float32),
        pltpu.VMEM((SLICE,), jnp.float32),
        pltpu.VMEM((128,), jnp.float32),
        pltpu.VMEM((WROWS, 128), jnp.int32),
        pltpu.VMEM((WROWS, 128), jnp.int32),
        pltpu.SemaphoreType.DMA,
    ],
)
def _deg_sc(col_hbm, dinv_hbm, deg_sh, dbuf, ones, cidxA, cidxB, ssem):
    s = lax.axis_index("s")
    c = lax.axis_index("c")
    r0 = SLICE * s
    ebase = s * (EDGES_PER_TILE // 128)

    def _widx(w):
        return ebase + w * WROWS

    def _za(i, carry):
        dbuf[pl.ds(i * 16, 16)] = jnp.zeros((16,), jnp.float32)
        return carry
    lax.fori_loop(0, SLICE // 16, _za, 0)
    for i in range(128 // 16):
        ones[pl.ds(i * 16, 16)] = jnp.ones((16,), jnp.float32)

    pltpu.sync_copy(dbuf, deg_sh.at[pl.ds(r0, SLICE)])
    plsc.subcore_barrier()

    pltpu.sync_copy(col_hbm.at[pl.ds(_widx(0), WROWS)], cidxA)

    def _deg_pair(k, carry):
        w0 = 2 * k
        sc0 = [pltpu.async_copy(ones, deg_sh.at[cidxA.at[j]], ssem, add=True)
               for j in range(WROWS)]
        pltpu.sync_copy(col_hbm.at[pl.ds(_widx(w0 + 1), WROWS)], cidxB)
        for cp in sc0:
            cp.wait()
        sc1 = [pltpu.async_copy(ones, deg_sh.at[cidxB.at[j]], ssem, add=True)
               for j in range(WROWS)]

        @pl.when(k < NPAIR - 1)
        def _():
            pltpu.sync_copy(col_hbm.at[pl.ds(_widx(w0 + 2), WROWS)], cidxA)
        for cp in sc1:
            cp.wait()
        return carry
    lax.fori_loop(0, NPAIR, _deg_pair, 0)
    plsc.subcore_barrier()

    pltpu.sync_copy(deg_sh.at[pl.ds(r0, SLICE)], dbuf)

    def _rsqrt(i, carry):
        d = dbuf[pl.ds(i * 16, 16)] + 1.0
        st = d
        for _ in range(15):
            st = 0.5 * (st + d / st)
        dbuf[pl.ds(i * 16, 16)] = 1.0 / st
        return carry
    lax.fori_loop(0, SLICE // 16, _rsqrt, 0)

    @pl.when(c == 0)
    def _():
        pltpu.sync_copy(dbuf, dinv_hbm.at[pl.ds(r0, SLICE)])


@functools.partial(
    pl.kernel,
    out_type=jax.ShapeDtypeStruct((N_NODES, D_OUT), jnp.float32),
    mesh=_MESH,
    compiler_params=pltpu.CompilerParams(use_tc_tiling_on_sc=False),
    scratch_types=[
        pltpu.VMEM_SHARED((N_PAD, HALF), jnp.float32),
        pltpu.VMEM_SHARED((N_PAD, HALF), jnp.float32),
        pltpu.VMEM((SLICE,), jnp.float32),
        pltpu.VMEM((WROWS, 128), jnp.int32),
        pltpu.VMEM((WROWS, 128), jnp.int32),
        pltpu.VMEM((WROWS, 128), jnp.int32),
        pltpu.VMEM((WROWS, 128), jnp.int32),
        pltpu.VMEM((WIN, HALF), jnp.float32),
        pltpu.VMEM((WIN, HALF), jnp.float32),
        pltpu.SemaphoreType.DMA,
        pltpu.SemaphoreType.DMA,
    ],
)
def _prop_sc(row_hbm, col_hbm, h0_hbm, dinv_hbm, out_hbm,
             g_sh, acc_sh, dbuf,
             ridxA, cidxA, ridxB, cidxB, gbufA, gbufB, gsem, ssem):
    s = lax.axis_index("s")
    c = lax.axis_index("c")
    r0 = SLICE * s
    ebase = s * (EDGES_PER_TILE // 128)
    wslc = gbufA.at[pl.ds(0, SLICE), :]
    wtail = gbufA.at[pl.ds(0, TAIL), :]

    def _widx(w):
        return ebase + w * WROWS

    pltpu.sync_copy(dinv_hbm.at[pl.ds(r0, SLICE)], dbuf)

    def _make_scale(square):
        def _scale(i, carry):
            dvec = dbuf[pl.ds(i * 16, 16)]
            if square:
                dvec = dvec * dvec
            for r in range(16):
                dv = jnp.full((16,), dvec[r])
                base = i * 16 + r
                wslc[base, pl.ds(0, 16)] = wslc[base, pl.ds(0, 16)] * dv
                wslc[base, pl.ds(16, 16)] = wslc[base, pl.ds(16, 16)] * dv
            return carry
        return _scale

    @pl.when(s < NSUB - 1)
    def _():
        pltpu.sync_copy(h0_hbm.at[c, pl.ds(r0, SLICE), :], wslc)

    @pl.when(s == NSUB - 1)
    def _():
        pltpu.sync_copy(h0_hbm.at[c, pl.ds(r0, TAIL), :], wtail)

    lax.fori_loop(0, SLICE // 16, _make_scale(False), 0)
    pltpu.sync_copy(wslc, g_sh.at[pl.ds(r0, SLICE)])
    pltpu.sync_copy(wslc, acc_sh.at[pl.ds(r0, SLICE)])
    plsc.subcore_barrier()

    def _fire_gathers(ridx, gbuf):
        for j in range(WROWS):
            pltpu.async_copy(g_sh.at[ridx.at[j]],
                             gbuf.at[pl.ds(j * 128, 128)], gsem)

    def _drain_gathers(ridx, gbuf):
        for j in range(WROWS):
            pltpu.make_async_copy(g_sh.at[ridx.at[j]],
                                  gbuf.at[pl.ds(j * 128, 128)], gsem).wait()

    def _load_idx(w, ridx, cidx):
        pltpu.sync_copy(row_hbm.at[pl.ds(_widx(w), WROWS)], ridx)
        pltpu.sync_copy(col_hbm.at[pl.ds(_widx(w), WROWS)], cidx)

    for rnd in range(2):
        _load_idx(0, ridxA, cidxA)
        _fire_gathers(ridxA, gbufA)

        def _pair(k, carry):
            w0 = 2 * k
            _load_idx(w0 + 1, ridxB, cidxB)
            _drain_gathers(ridxA, gbufA)
            sc0 = [pltpu.async_copy(gbufA.at[pl.ds(j * 128, 128)],
                                    acc_sh.at[cidxA.at[j]], ssem, add=True)
                   for j in range(WROWS)]
            _fire_gathers(ridxB, gbufB)
            for cp in sc0:
                cp.wait()

            @pl.when(k < NPAIR - 1)
            def _():
                _load_idx(w0 + 2, ridxA, cidxA)
            _drain_gathers(ridxB, gbufB)
            sc1 = [pltpu.async_copy(gbufB.at[pl.ds(j * 128, 128)],
                                    acc_sh.at[cidxB.at[j]], ssem, add=True)
                   for j in range(WROWS)]

            @pl.when(k < NPAIR - 1)
            def _():
                _fire_gathers(ridxA, gbufA)
            for cp in sc1:
                cp.wait()
            return carry
        lax.fori_loop(0, NPAIR, _pair, 0)
        plsc.subcore_barrier()

        pltpu.sync_copy(acc_sh.at[pl.ds(r0, SLICE)], wslc)
        if rnd == 0:
            lax.fori_loop(0, SLICE // 16, _make_scale(True), 0)
            pltpu.sync_copy(wslc, g_sh.at[pl.ds(r0, SLICE)])
            pltpu.sync_copy(wslc, acc_sh.at[pl.ds(r0, SLICE)])
            plsc.subcore_barrier()
        else:
            lax.fori_loop(0, SLICE // 16, _make_scale(False), 0)

            @pl.when(s < NSUB - 1)
            def _():
                pltpu.sync_copy(
                    wslc, out_hbm.at[pl.ds(r0, SLICE), pl.ds(c * HALF, HALF)])

            @pl.when(s == NSUB - 1)
            def _():
                pltpu.sync_copy(
                    wtail, out_hbm.at[pl.ds(r0, TAIL), pl.ds(c * HALF, HALF)])


def kernel(x, edges, W, b):
    row = edges[0].astype(jnp.int32)
    col = edges[1].astype(jnp.int32)
    pad_n = E_PAD - N_EDGES
    dump = N_NODES + (jnp.arange(pad_n, dtype=jnp.int32) % (N_PAD - N_NODES))
    row_p = jnp.concatenate([row, dump]).reshape(E_PAD // 128, 128)
    col_p = jnp.concatenate([col, dump]).reshape(E_PAD // 128, 128)

    h0 = _linear_tc(x, W, b)
    dinv = _deg_sc(col_p)
    return _prop_sc(row_p, col_p, h0, dinv)


# speedup vs baseline: 43.2142x; 1.0348x over previous
import functools

import jax
import jax.numpy as jnp
from jax import lax
from jax.experimental import pallas as pl
from jax.experimental.pallas import tpu as pltpu
from jax.experimental.pallas import tpu_sc as plsc

N_NODES = 10000
N_EDGES = 320000
D_IN = 128
D_OUT = 64
HALF = D_OUT // 2
NSC = 2
NSUB = 16
SLICE = 640
N_PAD = NSUB * SLICE
TAIL = N_NODES - (NSUB - 1) * SLICE
WIN = 1024
WROWS = WIN // 128
EPS = 20480
E15 = N_EDGES - (NSUB - 1) * EPS
NPAIR_FULL = EPS // WIN // 2
NWIN15 = E15 // WIN
NPAIR15 = NWIN15 // 2
TAILE = E15 - NWIN15 * WIN
TAILR = TAILE // 128
DPAIR = NPAIR_FULL // 2
DWIN15 = NWIN15 // 2
DPAIR15 = DWIN15 // 2
NB = 2000


def _mm_body(x_ref, w_ref, b_ref, o_ref):
    o_ref[0] = (
        lax.dot_general(x_ref[...], w_ref[0], (((1,), (1,)), ((), ())),
                        preferred_element_type=jnp.float32)
        + b_ref[0]
    )


def _linear_tc(x, W, b):
    return pl.pallas_call(
        _mm_body,
        grid=(NSC, N_NODES // NB),
        in_specs=[
            pl.BlockSpec((NB, D_IN), lambda c, n: (n, 0)),
            pl.BlockSpec((1, HALF, D_IN), lambda c, n: (c, 0, 0)),
            pl.BlockSpec((1, 1, HALF), lambda c, n: (c, 0, 0)),
        ],
        out_specs=pl.BlockSpec((1, NB, HALF), lambda c, n: (c, n, 0)),
        out_shape=jax.ShapeDtypeStruct((NSC, N_NODES, HALF), jnp.float32),
    )(x, W.reshape(NSC, HALF, D_IN), b.reshape(NSC, 1, HALF))


_MESH = plsc.VectorSubcoreMesh(
    core_axis_name="c", subcore_axis_name="s", num_cores=NSC, num_subcores=NSUB
)


@functools.partial(
    pl.kernel,
    out_type=jax.ShapeDtypeStruct((NSC, N_PAD), jnp.float32),
    mesh=_MESH,
    compiler_params=pltpu.CompilerParams(use_tc_tiling_on_sc=False),
    scratch_types=[
        pltpu.VMEM_SHARED((N_PAD,), jnp.float32),
        pltpu.VMEM((SLICE,), jnp.float32),
        pltpu.VMEM((128,), jnp.float32),
        pltpu.VMEM((WIN,), jnp.int32),
        pltpu.VMEM((WIN,), jnp.int32),
        pltpu.SemaphoreType.DMA,
    ],
)
def _deg_sc(edges_hbm, deg_hbm, deg_sh, dbuf, ones, cidxA, cidxB, ssem):
    s = lax.axis_index("s")
    c = lax.axis_index("c")
    r0 = SLICE * s
    ebase = s * EPS

    def _za(i, carry):
        dbuf[pl.ds(i * 16, 16)] = jnp.zeros((16,), jnp.float32)
        return carry
    lax.fori_loop(0, SLICE // 16, _za, 0)
    for i in range(128 // 16):
        ones[pl.ds(i * 16, 16)] = jnp.ones((16,), jnp.float32)

    pltpu.sync_copy(dbuf, deg_sh.at[pl.ds(r0, SLICE)])
    plsc.subcore_barrier()

    def _load_col(w, cidx):
        pltpu.sync_copy(edges_hbm.at[1, pl.ds(ebase + w * WIN, WIN)], cidx)

    def _fire_adds(cidx, nrows=WROWS):
        return [pltpu.async_copy(
                    ones, deg_sh.at[cidx.at[pl.ds(j * 128, 128)]],
                    ssem, add=True)
                for j in range(nrows)]

    def _deg_pipeline(wbase, npair):
        _load_col(wbase, cidxA)

        def _pair(k, carry):
            w0 = wbase + 2 * k
            sc0 = _fire_adds(cidxA)
            _load_col(w0 + 1, cidxB)
            for cp in sc0:
                cp.wait()
            sc1 = _fire_adds(cidxB)

            @pl.when(k < npair - 1)
            def _():
                _load_col(w0 + 2, cidxA)
            for cp in sc1:
                cp.wait()
            return carry
        lax.fori_loop(0, npair, _pair, 0)

    @pl.when(s < NSUB - 1)
    def _():
        _deg_pipeline(DPAIR * 2 * c, DPAIR)

    @pl.when(s == NSUB - 1)
    def _():
        _deg_pipeline(DWIN15 * c, DPAIR15)

        @pl.when(c == 1)
        def _():
            pltpu.sync_copy(
                edges_hbm.at[1, pl.ds(ebase + NWIN15 * WIN, TAILE)],
                cidxA.at[pl.ds(0, TAILE)])
            for cp in _fire_adds(cidxA, TAILR):
                cp.wait()

    plsc.subcore_barrier()
    pltpu.sync_copy(deg_sh.at[pl.ds(r0, SLICE)], deg_hbm.at[c, pl.ds(r0, SLICE)])


@functools.partial(
    pl.kernel,
    out_type=jax.ShapeDtypeStruct((N_NODES, D_OUT), jnp.float32),
    mesh=_MESH,
    compiler_params=pltpu.CompilerParams(use_tc_tiling_on_sc=False),
    scratch_types=[
        pltpu.VMEM_SHARED((N_PAD, HALF), jnp.float32),
        pltpu.VMEM_SHARED((N_PAD, HALF), jnp.float32),
        pltpu.VMEM((SLICE,), jnp.float32),
        pltpu.VMEM((SLICE,), jnp.float32),
        pltpu.VMEM((WIN,), jnp.int32),
        pltpu.VMEM((WIN,), jnp.int32),
        pltpu.VMEM((WIN,), jnp.int32),
        pltpu.VMEM((WIN,), jnp.int32),
        pltpu.VMEM((WIN, HALF), jnp.float32),
        pltpu.VMEM((WIN, HALF), jnp.float32),
        pltpu.SemaphoreType.DMA,
        pltpu.SemaphoreType.DMA,
    ],
)
def _prop_sc(edges_hbm, h0_hbm, deg_hbm, out_hbm,
             g_sh, acc_sh, dbuf, dbuf2,
             ridxA, cidxA, ridxB, cidxB, gbufA, gbufB, gsem, ssem):
    s = lax.axis_index("s")
    c = lax.axis_index("c")
    r0 = SLICE * s
    ebase = s * EPS
    wslc = gbufA.at[pl.ds(0, SLICE), :]
    wtail = gbufA.at[pl.ds(0, TAIL), :]

    pltpu.sync_copy(deg_hbm.at[0, pl.ds(r0, SLICE)], dbuf)
    pltpu.sync_copy(deg_hbm.at[1, pl.ds(r0, SLICE)], dbuf2)

    def _rsqrt(i, carry):
        d = dbuf[pl.ds(i * 16, 16)] + dbuf2[pl.ds(i * 16, 16)] + 1.0
        st = d
        for _ in range(15):
            st = 0.5 * (st + d / st)
        dbuf[pl.ds(i * 16, 16)] = 1.0 / st
        return carry
    lax.fori_loop(0, SLICE // 16, _rsqrt, 0)

    def _make_scale(square):
        def _scale(i, carry):
            dvec = dbuf[pl.ds(i * 16, 16)]
            if square:
                dvec = dvec * dvec
            for r in range(16):
                dv = jnp.full((16,), dvec[r])
                base = i * 16 + r
                wslc[base, pl.ds(0, 16)] = wslc[base, pl.ds(0, 16)] * dv
                wslc[base, pl.ds(16, 16)] = wslc[base, pl.ds(16, 16)] * dv
            return carry
        return _scale

    @pl.when(s < NSUB - 1)
    def _():
        pltpu.sync_copy(h0_hbm.at[c, pl.ds(r0, SLICE), :], wslc)

    @pl.when(s == NSUB - 1)
    def _():
        pltpu.sync_copy(h0_hbm.at[c, pl.ds(r0, TAIL), :], wtail)

    lax.fori_loop(0, SLICE // 16, _make_scale(False), 0)
    pltpu.sync_copy(wslc, g_sh.at[pl.ds(r0, SLICE)])
    pltpu.sync_copy(wslc, acc_sh.at[pl.ds(r0, SLICE)])
    plsc.subcore_barrier()

    def _fire_gathers(ridx, gbuf, nrows=WROWS):
        for j in range(nrows):
            pltpu.async_copy(g_sh.at[ridx.at[pl.ds(j * 128, 128)]],
                             gbuf.at[pl.ds(j * 128, 128)], gsem)

    def _drain_gathers(ridx, gbuf, nrows=WROWS):
        for j in range(nrows):
            pltpu.make_async_copy(g_sh.at[ridx.at[pl.ds(j * 128, 128)]],
                                  gbuf.at[pl.ds(j * 128, 128)], gsem).wait()

    def _fire_scatters(gbuf, cidx, nrows=WROWS):
        return [pltpu.async_copy(gbuf.at[pl.ds(j * 128, 128)],
                                 acc_sh.at[cidx.at[pl.ds(j * 128, 128)]],
                                 ssem, add=True)
                for j in range(nrows)]

    def _load_idx(w, ridx, cidx):
        pltpu.sync_copy(edges_hbm.at[0, pl.ds(ebase + w * WIN, WIN)], ridx)
        pltpu.sync_copy(edges_hbm.at[1, pl.ds(ebase + w * WIN, WIN)], cidx)

    def _round_pipeline(npair):
        _load_idx(0, ridxA, cidxA)
        _fire_gathers(ridxA, gbufA)

        def _pair(k, carry):
            w0 = 2 * k
            _load_idx(w0 + 1, ridxB, cidxB)
            _drain_gathers(ridxA, gbufA)
            sc0 = _fire_scatters(gbufA, cidxA)
            _fire_gathers(ridxB, gbufB)
            for cp in sc0:
                cp.wait()

            @pl.when(k < npair - 1)
            def _():
                _load_idx(w0 + 2, ridxA, cidxA)
            _drain_gathers(ridxB, gbufB)
            sc1 = _fire_scatters(gbufB, cidxB)

            @pl.when(k < npair - 1)
            def _():
                _fire_gathers(ridxA, gbufA)
            for cp in sc1:
                cp.wait()
            return carry
        lax.fori_loop(0, npair, _pair, 0)

    def _round_tail():
        e0 = ebase + NWIN15 * WIN
        pltpu.sync_copy(edges_hbm.at[0, pl.ds(e0, TAILE)],
                        ridxA.at[pl.ds(0, TAILE)])
        pltpu.sync_copy(edges_hbm.at[1, pl.ds(e0, TAILE)],
                        cidxA.at[pl.ds(0, TAILE)])
        _fire_gathers(ridxA, gbufA, TAILR)
        _drain_gathers(ridxA, gbufA, TAILR)
        for cp in _fire_scatters(gbufA, cidxA, TAILR):
            cp.wait()

    for rnd in range(2):
        @pl.when(s < NSUB - 1)
        def _():
            _round_pipeline(NPAIR_FULL)

        @pl.when(s == NSUB - 1)
        def _():
            _round_pipeline(NPAIR15)
            _round_tail()

        plsc.subcore_barrier()

        pltpu.sync_copy(acc_sh.at[pl.ds(r0, SLICE)], wslc)
        if rnd == 0:
            lax.fori_loop(0, SLICE // 16, _make_scale(True), 0)
            pltpu.sync_copy(wslc, g_sh.at[pl.ds(r0, SLICE)])
            pltpu.sync_copy(wslc, acc_sh.at[pl.ds(r0, SLICE)])
            plsc.subcore_barrier()
        else:
            lax.fori_loop(0, SLICE // 16, _make_scale(False), 0)

            @pl.when(s < NSUB - 1)
            def _():
                pltpu.sync_copy(
                    wslc, out_hbm.at[pl.ds(r0, SLICE), pl.ds(c * HALF, HALF)])

            @pl.when(s == NSUB - 1)
            def _():
                pltpu.sync_copy(
                    wtail, out_hbm.at[pl.ds(r0, TAIL), pl.ds(c * HALF, HALF)])


def kernel(x, edges, W, b):
    e32 = edges.astype(jnp.int32)
    h0 = _linear_tc(x, W, b)
    deg = _deg_sc(e32)
    return _prop_sc(e32, h0, deg)


# speedup vs baseline: 43.7434x; 1.0122x over previous
import functools

import jax
import jax.numpy as jnp
from jax import lax
from jax.experimental import pallas as pl
from jax.experimental.pallas import tpu as pltpu
from jax.experimental.pallas import tpu_sc as plsc

N_NODES = 10000
N_EDGES = 320000
D_IN = 128
D_OUT = 64
HALF = D_OUT // 2
NSC = 2
NSUB = 16
SLICE = 640
N_PAD = NSUB * SLICE
TAIL = N_NODES - (NSUB - 1) * SLICE
WIN = 1024
WROWS = WIN // 128
EPS = 20480
E15 = N_EDGES - (NSUB - 1) * EPS
NPAIR_FULL = EPS // WIN // 2
NWIN15 = E15 // WIN
NPAIR15 = NWIN15 // 2
TAILE = E15 - NWIN15 * WIN
TAILR = TAILE // 128
DPAIR = NPAIR_FULL // 2
DWIN15 = NWIN15 // 2
DPAIR15 = DWIN15 // 2
NB = 2000


def _mm_body(x_ref, w_ref, b_ref, o_ref):
    o_ref[...] = (
        lax.dot_general(x_ref[...], w_ref[...], (((1,), (1,)), ((), ())),
                        preferred_element_type=jnp.float32)
        + b_ref[...]
    )


def _linear_tc(x, W, b):
    return pl.pallas_call(
        _mm_body,
        grid=(N_NODES // NB,),
        in_specs=[
            pl.BlockSpec((NB, D_IN), lambda n: (n, 0)),
            pl.BlockSpec((D_OUT, D_IN), lambda n: (0, 0)),
            pl.BlockSpec((1, D_OUT), lambda n: (0, 0)),
        ],
        out_specs=pl.BlockSpec((NB, D_OUT), lambda n: (n, 0)),
        out_shape=jax.ShapeDtypeStruct((N_NODES, D_OUT), jnp.float32),
    )(x, W, b.reshape(1, D_OUT))


_MESH = plsc.VectorSubcoreMesh(
    core_axis_name="c", subcore_axis_name="s", num_cores=NSC, num_subcores=NSUB
)


@functools.partial(
    pl.kernel,
    out_type=jax.ShapeDtypeStruct((NSC, N_PAD), jnp.float32),
    mesh=_MESH,
    compiler_params=pltpu.CompilerParams(use_tc_tiling_on_sc=False),
    scratch_types=[
        pltpu.VMEM_SHARED((N_PAD,), jnp.float32),
        pltpu.VMEM((SLICE,), jnp.float32),
        pltpu.VMEM((128,), jnp.float32),
        pltpu.VMEM((WIN,), jnp.int32),
        pltpu.VMEM((WIN,), jnp.int32),
        pltpu.SemaphoreType.DMA,
    ],
)
def _deg_sc(col_hbm, deg_hbm, deg_sh, dbuf, ones, cidxA, cidxB, ssem):
    s = lax.axis_index("s")
    c = lax.axis_index("c")
    r0 = SLICE * s
    ebase = s * EPS

    def _za(i, carry):
        dbuf[pl.ds(i * 16, 16)] = jnp.zeros((16,), jnp.float32)
        return carry
    lax.fori_loop(0, SLICE // 16, _za, 0)
    for i in range(128 // 16):
        ones[pl.ds(i * 16, 16)] = jnp.ones((16,), jnp.float32)

    pltpu.sync_copy(dbuf, deg_sh.at[pl.ds(r0, SLICE)])
    plsc.subcore_barrier()

    def _load_col(w, cidx):
        pltpu.sync_copy(col_hbm.at[pl.ds(ebase + w * WIN, WIN)], cidx)

    def _fire_adds(cidx, nrows=WROWS):
        return [pltpu.async_copy(
                    ones, deg_sh.at[cidx.at[pl.ds(j * 128, 128)]],
                    ssem, add=True)
                for j in range(nrows)]

    def _deg_pipeline(wbase, npair):
        _load_col(wbase, cidxA)

        def _pair(k, carry):
            w0 = wbase + 2 * k
            sc0 = _fire_adds(cidxA)
            _load_col(w0 + 1, cidxB)
            for cp in sc0:
                cp.wait()
            sc1 = _fire_adds(cidxB)

            @pl.when(k < npair - 1)
            def _():
                _load_col(w0 + 2, cidxA)
            for cp in sc1:
                cp.wait()
            return carry
        lax.fori_loop(0, npair, _pair, 0)

    @pl.when(s < NSUB - 1)
    def _():
        _deg_pipeline(DPAIR * 2 * c, DPAIR)

    @pl.when(s == NSUB - 1)
    def _():
        _deg_pipeline(DWIN15 * c, DPAIR15)

        @pl.when(c == 1)
        def _():
            pltpu.sync_copy(col_hbm.at[pl.ds(ebase + NWIN15 * WIN, TAILE)],
                            cidxA.at[pl.ds(0, TAILE)])
            for cp in _fire_adds(cidxA, TAILR):
                cp.wait()

    plsc.subcore_barrier()
    pltpu.sync_copy(deg_sh.at[pl.ds(r0, SLICE)], deg_hbm.at[c, pl.ds(r0, SLICE)])


@functools.partial(
    pl.kernel,
    out_type=jax.ShapeDtypeStruct((N_NODES, D_OUT), jnp.float32),
    mesh=_MESH,
    compiler_params=pltpu.CompilerParams(use_tc_tiling_on_sc=False),
    scratch_types=[
        pltpu.VMEM_SHARED((N_PAD, HALF), jnp.float32),
        pltpu.VMEM_SHARED((N_PAD, HALF), jnp.float32),
        pltpu.VMEM((SLICE,), jnp.float32),
        pltpu.VMEM((SLICE,), jnp.float32),
        pltpu.VMEM((WIN,), jnp.int32),
        pltpu.VMEM((WIN,), jnp.int32),
        pltpu.VMEM((WIN,), jnp.int32),
        pltpu.VMEM((WIN,), jnp.int32),
        pltpu.VMEM((WIN, HALF), jnp.float32),
        pltpu.VMEM((WIN, HALF), jnp.float32),
        pltpu.SemaphoreType.DMA,
        pltpu.SemaphoreType.DMA,
    ],
)
def _prop_sc(row_hbm, col_hbm, h0_hbm, deg_hbm, out_hbm,
             g_sh, acc_sh, dbuf, dbuf2,
             ridxA, cidxA, ridxB, cidxB, gbufA, gbufB, gsem, ssem):
    s = lax.axis_index("s")
    c = lax.axis_index("c")
    r0 = SLICE * s
    ebase = s * EPS
    wslc = gbufA.at[pl.ds(0, SLICE), :]
    wtail = gbufA.at[pl.ds(0, TAIL), :]

    pltpu.sync_copy(deg_hbm.at[0, pl.ds(r0, SLICE)], dbuf)
    pltpu.sync_copy(deg_hbm.at[1, pl.ds(r0, SLICE)], dbuf2)

    def _dinv_vec(i, carry):
        y = dbuf[pl.ds(i * 16, 16)] + dbuf2[pl.ds(i * 16, 16)] + 1.0
        x = jnp.full((16,), 1.0, jnp.float32)
        for t in (2.0, 8.0, 32.0, 128.0, 512.0,
                  2048.0, 8192.0, 32768.0, 131072.0):
            x = jnp.where(y >= t, x * 0.5, x)
        h = y * 0.5
        for _ in range(6):
            x = x * (1.5 - h * x * x)
        dbuf[pl.ds(i * 16, 16)] = x
        return carry
    lax.fori_loop(0, SLICE // 16, _dinv_vec, 0)

    def _make_scale(square):
        def _scale(i, carry):
            dvec = dbuf[pl.ds(i * 16, 16)]
            if square:
                dvec = dvec * dvec
            for r in range(16):
                dv = jnp.full((16,), dvec[r])
                base = i * 16 + r
                wslc[base, pl.ds(0, 16)] = wslc[base, pl.ds(0, 16)] * dv
                wslc[base, pl.ds(16, 16)] = wslc[base, pl.ds(16, 16)] * dv
            return carry
        return _scale

    @pl.when(s < NSUB - 1)
    def _():
        pltpu.sync_copy(
            h0_hbm.at[pl.ds(r0, SLICE), pl.ds(c * HALF, HALF)], wslc)

    @pl.when(s == NSUB - 1)
    def _():
        pltpu.sync_copy(
            h0_hbm.at[pl.ds(r0, TAIL), pl.ds(c * HALF, HALF)], wtail)

    lax.fori_loop(0, SLICE // 16, _make_scale(False), 0)
    pltpu.sync_copy(wslc, g_sh.at[pl.ds(r0, SLICE)])
    pltpu.sync_copy(wslc, acc_sh.at[pl.ds(r0, SLICE)])
    plsc.subcore_barrier()

    def _fire_gathers(ridx, gbuf, nrows=WROWS):
        for j in range(nrows):
            pltpu.async_copy(g_sh.at[ridx.at[pl.ds(j * 128, 128)]],
                             gbuf.at[pl.ds(j * 128, 128)], gsem)

    def _drain_gathers(ridx, gbuf, nrows=WROWS):
        for j in range(nrows):
            pltpu.make_async_copy(g_sh.at[ridx.at[pl.ds(j * 128, 128)]],
                                  gbuf.at[pl.ds(j * 128, 128)], gsem).wait()

    def _fire_scatters(gbuf, cidx, nrows=WROWS):
        return [pltpu.async_copy(gbuf.at[pl.ds(j * 128, 128)],
                                 acc_sh.at[cidx.at[pl.ds(j * 128, 128)]],
                                 ssem, add=True)
                for j in range(nrows)]

    def _load_idx(w, ridx, cidx):
        pltpu.sync_copy(row_hbm.at[pl.ds(ebase + w * WIN, WIN)], ridx)
        pltpu.sync_copy(col_hbm.at[pl.ds(ebase + w * WIN, WIN)], cidx)

    def _round_pipeline(npair):
        _load_idx(0, ridxA, cidxA)
        _fire_gathers(ridxA, gbufA)

        def _pair(k, carry):
            w0 = 2 * k
            _load_idx(w0 + 1, ridxB, cidxB)
            _drain_gathers(ridxA, gbufA)
            sc0 = _fire_scatters(gbufA, cidxA)
            _fire_gathers(ridxB, gbufB)
            for cp in sc0:
                cp.wait()

            @pl.when(k < npair - 1)
            def _():
                _load_idx(w0 + 2, ridxA, cidxA)
            _drain_gathers(ridxB, gbufB)
            sc1 = _fire_scatters(gbufB, cidxB)

            @pl.when(k < npair - 1)
            def _():
                _fire_gathers(ridxA, gbufA)
            for cp in sc1:
                cp.wait()
            return carry
        lax.fori_loop(0, npair, _pair, 0)

    def _round_tail():
        e0 = ebase + NWIN15 * WIN
        pltpu.sync_copy(row_hbm.at[pl.ds(e0, TAILE)],
                        ridxA.at[pl.ds(0, TAILE)])
        pltpu.sync_copy(col_hbm.at[pl.ds(e0, TAILE)],
                        cidxA.at[pl.ds(0, TAILE)])
        _fire_gathers(ridxA, gbufA, TAILR)
        _drain_gathers(ridxA, gbufA, TAILR)
        for cp in _fire_scatters(gbufA, cidxA, TAILR):
            cp.wait()

    for rnd in range(2):
        @pl.when(s < NSUB - 1)
        def _():
            _round_pipeline(NPAIR_FULL)

        @pl.when(s == NSUB - 1)
        def _():
            _round_pipeline(NPAIR15)
            _round_tail()

        plsc.subcore_barrier()

        pltpu.sync_copy(acc_sh.at[pl.ds(r0, SLICE)], wslc)
        if rnd == 0:
            lax.fori_loop(0, SLICE // 16, _make_scale(True), 0)
            pltpu.sync_copy(wslc, g_sh.at[pl.ds(r0, SLICE)])
            pltpu.sync_copy(wslc, acc_sh.at[pl.ds(r0, SLICE)])
            plsc.subcore_barrier()
        else:
            lax.fori_loop(0, SLICE // 16, _make_scale(False), 0)

            @pl.when(s < NSUB - 1)
            def _():
                pltpu.sync_copy(
                    wslc, out_hbm.at[pl.ds(r0, SLICE), pl.ds(c * HALF, HALF)])

            @pl.when(s == NSUB - 1)
            def _():
                pltpu.sync_copy(
                    wtail, out_hbm.at[pl.ds(r0, TAIL), pl.ds(c * HALF, HALF)])


def kernel(x, edges, W, b):
    e32 = edges.astype(jnp.int32)
    row = e32[0]
    col = e32[1]
    h0 = _linear_tc(x, W, b)
    deg = _deg_sc(col)
    return _prop_sc(row, col, h0, deg)
